# baseline (speedup 1.0000x reference)
import functools

import jax
import jax.numpy as jnp
from jax import lax
from jax.experimental import pallas as pl
from jax.experimental.pallas import tpu as pltpu
from jax.experimental.pallas import tpu_sc as plsc

D = 2048
H = 512
E = 64
N = 32


def _gate_kernel(x_ref, gw_ref, eid_ref, p_ref):
    x = x_ref[...]
    gw = gw_ref[...]
    s = jax.lax.dot_general(x, gw, (((1,), (1,)), ((), ())),
                            preferred_element_type=jnp.float32)
    m = jnp.max(s, axis=1, keepdims=True)
    p_top = 1.0 / jnp.sum(jnp.exp(s - m), axis=1, keepdims=True)
    col = jax.lax.broadcasted_iota(jnp.int32, (N, E), 1)
    e_id = jnp.min(jnp.where(s == m, col, E), axis=1)
    eid_ref[0, :] = e_id
    p_ref[0, :] = p_top[:, 0]


def _route_sc_kernel(eid_hbm, p_hbm, se_hbm, st_hbm, sp_hbm,
                     eid_v, p_v, se_v, st_v, sp_v):
    wid = lax.axis_index("s") * 2 + lax.axis_index("c")

    @pl.when(wid == 0)
    def _():
        pltpu.sync_copy(eid_hbm, eid_v)
        pltpu.sync_copy(p_hbm, p_v)
        e0 = eid_v[pl.ds(0, 16)]
        e1 = eid_v[pl.ds(16, 16)]
        io = lax.iota(jnp.int32, 16)
        key0 = e0 * N + io
        key1 = e1 * N + io + 16
        r0 = jnp.zeros((16,), jnp.int32)
        r1 = jnp.zeros((16,), jnp.int32)
        for lane in range(16):
            kb0 = jnp.full((16,), key0[lane], jnp.int32)
            c0 = (plsc.all_reduce_population_count(key0 < kb0)
                  + plsc.all_reduce_population_count(key1 < kb0))
            r0 = jnp.where(io == lane, c0, r0)
            kb1 = jnp.full((16,), key1[lane], jnp.int32)
            c1 = (plsc.all_reduce_population_count(key0 < kb1)
                  + plsc.all_reduce_population_count(key1 < kb1))
            r1 = jnp.where(io == lane, c1, r1)
        p0 = p_v[pl.ds(0, 16)]
        p1 = p_v[pl.ds(16, 16)]
        plsc.store_scatter(se_v, [r0], e0)
        plsc.store_scatter(se_v, [r1], e1)
        plsc.store_scatter(st_v, [r0], io)
        plsc.store_scatter(st_v, [r1], io + 16)
        plsc.store_scatter(sp_v, [r0], p0)
        plsc.store_scatter(sp_v, [r1], p1)
        pltpu.sync_copy(se_v, se_hbm)
        pltpu.sync_copy(st_v, st_hbm)
        pltpu.sync_copy(sp_v, sp_hbm)


_route_sc = functools.partial(
    pl.kernel,
    out_type=(
        jax.ShapeDtypeStruct((N,), jnp.int32),
        jax.ShapeDtypeStruct((N,), jnp.int32),
        jax.ShapeDtypeStruct((N,), jnp.float32),
    ),
    mesh=plsc.VectorSubcoreMesh(core_axis_name="c", subcore_axis_name="s",
                                num_cores=1, num_subcores=1),
    compiler_params=pltpu.CompilerParams(needs_layout_passes=False),
    scratch_types=[
        pltpu.VMEM((N,), jnp.int32),
        pltpu.VMEM((N,), jnp.float32),
        pltpu.VMEM((N,), jnp.int32),
        pltpu.VMEM((N,), jnp.int32),
        pltpu.VMEM((N,), jnp.float32),
    ],
)(_route_sc_kernel)


HB = 128
NJ = H // HB


def _ffn_kernel(se_ref, st_ref, x_ref, w1_ref, w3_ref, w2_ref, p_ref, o_ref):
    del se_ref, st_ref
    j = pl.program_id(1)
    x = x_ref[0]
    w1 = w1_ref[0]
    w3 = w3_ref[0]
    w2 = w2_ref[0]
    h1 = jax.lax.dot_general(x, w1, (((1,), (1,)), ((), ())),
                             preferred_element_type=jnp.float32)
    h3 = jax.lax.dot_general(x, w3, (((1,), (1,)), ((), ())),
                             preferred_element_type=jnp.float32)
    h = jax.nn.silu(h1) * h3
    y = jax.lax.dot_general(h, w2, (((1,), (1,)), ((), ())),
                            preferred_element_type=jnp.float32)
    y = y * p_ref[0, 0, 0]

    @pl.when(j == 0)
    def _():
        o_ref[0] = y

    @pl.when(j != 0)
    def _():
        o_ref[0] += y


def kernel(x, gate_w, w1, w2, w3):
    orig_shape = x.shape
    xf = x.reshape(-1, orig_shape[-1])

    eid, p = pl.pallas_call(
        _gate_kernel,
        out_shape=(
            jax.ShapeDtypeStruct((1, N), jnp.int32),
            jax.ShapeDtypeStruct((1, N), jnp.float32),
        ),
    )(xf, gate_w)

    se1, st1, sp = _route_sc(eid.reshape(N), p.reshape(N))

    spv = sp.reshape(N, 1, 1)
    x3 = xf.reshape(N, 1, D)

    def _jj(i, j):
        return jnp.where(i % 2 == 0, j, NJ - 1 - j)

    grid_spec = pltpu.PrefetchScalarGridSpec(
        num_scalar_prefetch=2,
        grid=(N, NJ),
        in_specs=[
            pl.BlockSpec((1, 1, D), lambda i, j, se_r, st_r: (st_r[i], 0, 0)),
            pl.BlockSpec((1, HB, D),
                         lambda i, j, se_r, st_r: (se_r[i], _jj(i, j), 0)),
            pl.BlockSpec((1, HB, D),
                         lambda i, j, se_r, st_r: (se_r[i], _jj(i, j), 0)),
            pl.BlockSpec((1, D, HB),
                         lambda i, j, se_r, st_r: (se_r[i], 0, _jj(i, j))),
            pl.BlockSpec((1, 1, 1), lambda i, j, se_r, st_r: (i, 0, 0)),
        ],
        out_specs=pl.BlockSpec((1, 1, D),
                               lambda i, j, se_r, st_r: (st_r[i], 0, 0)),
    )

    y = pl.pallas_call(
        _ffn_kernel,
        grid_spec=grid_spec,
        out_shape=jax.ShapeDtypeStruct((N, 1, D), jnp.float32),
        compiler_params=pltpu.CompilerParams(
            vmem_limit_bytes=100 * 1024 * 1024,
        ),
    )(se1, st1, x3, w1, w3, w2, spv)

    return y.reshape(orig_shape)


# speedup vs baseline: 1.4471x; 1.4471x over previous
import functools

import jax
import jax.numpy as jnp
from jax import lax
from jax.experimental import pallas as pl
from jax.experimental.pallas import tpu as pltpu
from jax.experimental.pallas import tpu_sc as plsc

D = 2048
H = 512
E = 64
N = 32


def _gate_kernel(x_ref, gw_ref, eid_ref, p_ref):
    x = x_ref[...]
    gw = gw_ref[...]
    s = jax.lax.dot_general(x, gw, (((1,), (1,)), ((), ())),
                            preferred_element_type=jnp.float32)
    m = jnp.max(s, axis=1, keepdims=True)
    p_top = 1.0 / jnp.sum(jnp.exp(s - m), axis=1, keepdims=True)
    col = jax.lax.broadcasted_iota(jnp.int32, (N, E), 1)
    e_id = jnp.min(jnp.where(s == m, col, E), axis=1)
    eid_ref[0, :] = e_id
    p_ref[0, :] = p_top[:, 0]


def _route_sc_kernel(eid_hbm, p_hbm, se_hbm, st_hbm, sp_hbm,
                     eid_v, p_v, se_v, st_v, sp_v):
    wid = lax.axis_index("s") * 2 + lax.axis_index("c")

    @pl.when(wid == 0)
    def _():
        pltpu.sync_copy(eid_hbm, eid_v)
        pltpu.sync_copy(p_hbm, p_v)
        e0 = eid_v[pl.ds(0, 16)]
        e1 = eid_v[pl.ds(16, 16)]
        io = lax.iota(jnp.int32, 16)
        key0 = e0 * N + io
        key1 = e1 * N + io + 16
        r0 = jnp.zeros((16,), jnp.int32)
        r1 = jnp.zeros((16,), jnp.int32)
        for lane in range(16):
            kb0 = jnp.full((16,), key0[lane], jnp.int32)
            c0 = (plsc.all_reduce_population_count(key0 < kb0)
                  + plsc.all_reduce_population_count(key1 < kb0))
            r0 = jnp.where(io == lane, c0, r0)
            kb1 = jnp.full((16,), key1[lane], jnp.int32)
            c1 = (plsc.all_reduce_population_count(key0 < kb1)
                  + plsc.all_reduce_population_count(key1 < kb1))
            r1 = jnp.where(io == lane, c1, r1)
        p0 = p_v[pl.ds(0, 16)]
        p1 = p_v[pl.ds(16, 16)]
        plsc.store_scatter(se_v, [r0], e0)
        plsc.store_scatter(se_v, [r1], e1)
        plsc.store_scatter(st_v, [r0], io)
        plsc.store_scatter(st_v, [r1], io + 16)
        plsc.store_scatter(sp_v, [r0], p0)
        plsc.store_scatter(sp_v, [r1], p1)
        pltpu.sync_copy(se_v, se_hbm)
        pltpu.sync_copy(st_v, st_hbm)
        pltpu.sync_copy(sp_v, sp_hbm)


_route_sc = functools.partial(
    pl.kernel,
    out_type=(
        jax.ShapeDtypeStruct((N,), jnp.int32),
        jax.ShapeDtypeStruct((N,), jnp.int32),
        jax.ShapeDtypeStruct((N,), jnp.float32),
    ),
    mesh=plsc.VectorSubcoreMesh(core_axis_name="c", subcore_axis_name="s",
                                num_cores=1, num_subcores=1),
    compiler_params=pltpu.CompilerParams(needs_layout_passes=False),
    scratch_types=[
        pltpu.VMEM((N,), jnp.int32),
        pltpu.VMEM((N,), jnp.float32),
        pltpu.VMEM((N,), jnp.int32),
        pltpu.VMEM((N,), jnp.int32),
        pltpu.VMEM((N,), jnp.float32),
    ],
)(_route_sc_kernel)


def _ffn_kernel(se_ref, st_ref, x_ref, w1_ref, w3_ref, w2_ref, p_ref, o_ref):
    del se_ref, st_ref
    x = x_ref[0]
    w1 = w1_ref[0]
    w3 = w3_ref[0]
    w2 = w2_ref[0]
    h1 = jax.lax.dot_general(x, w1, (((1,), (1,)), ((), ())),
                             preferred_element_type=jnp.float32)
    h3 = jax.lax.dot_general(x, w3, (((1,), (1,)), ((), ())),
                             preferred_element_type=jnp.float32)
    h = jax.nn.silu(h1) * h3
    y = jax.lax.dot_general(h, w2, (((1,), (1,)), ((), ())),
                            preferred_element_type=jnp.float32)
    o_ref[0] = y * p_ref[0, 0, 0]


def kernel(x, gate_w, w1, w2, w3):
    orig_shape = x.shape
    xf = x.reshape(-1, orig_shape[-1])

    eid, p = pl.pallas_call(
        _gate_kernel,
        out_shape=(
            jax.ShapeDtypeStruct((1, N), jnp.int32),
            jax.ShapeDtypeStruct((1, N), jnp.float32),
        ),
    )(xf, gate_w)

    se1, st1, sp = _route_sc(eid.reshape(N), p.reshape(N))

    spv = sp.reshape(N, 1, 1)
    x3 = xf.reshape(N, 1, D)

    grid_spec = pltpu.PrefetchScalarGridSpec(
        num_scalar_prefetch=2,
        grid=(N,),
        in_specs=[
            pl.BlockSpec((1, 1, D), lambda i, se_r, st_r: (st_r[i], 0, 0)),
            pl.BlockSpec((1, H, D), lambda i, se_r, st_r: (se_r[i], 0, 0)),
            pl.BlockSpec((1, H, D), lambda i, se_r, st_r: (se_r[i], 0, 0)),
            pl.BlockSpec((1, D, H), lambda i, se_r, st_r: (se_r[i], 0, 0)),
            pl.BlockSpec((1, 1, 1), lambda i, se_r, st_r: (i, 0, 0)),
        ],
        out_specs=pl.BlockSpec((1, 1, D), lambda i, se_r, st_r: (st_r[i], 0, 0)),
    )

    y = pl.pallas_call(
        _ffn_kernel,
        grid_spec=grid_spec,
        out_shape=jax.ShapeDtypeStruct((N, 1, D), jnp.float32),
        compiler_params=pltpu.CompilerParams(
            vmem_limit_bytes=100 * 1024 * 1024,
        ),
    )(se1, st1, x3, w1, w3, w2, spv)

    return y.reshape(orig_shape)


# speedup vs baseline: 1.4669x; 1.0137x over previous
import functools

import jax
import jax.numpy as jnp
from jax import lax
from jax.experimental import pallas as pl
from jax.experimental.pallas import tpu as pltpu
from jax.experimental.pallas import tpu_sc as plsc

D = 2048
H = 512
E = 64
N = 32


def _gate_kernel(x_ref, gw_ref, eid_ref, p_ref):
    x = x_ref[...]
    gw = gw_ref[...]
    s = jax.lax.dot_general(x, gw, (((1,), (1,)), ((), ())),
                            preferred_element_type=jnp.float32)
    m = jnp.max(s, axis=1, keepdims=True)
    p_top = 1.0 / jnp.sum(jnp.exp(s - m), axis=1, keepdims=True)
    col = jax.lax.broadcasted_iota(jnp.int32, (N, E), 1)
    e_id = jnp.min(jnp.where(s == m, col, E), axis=1)
    eid_ref[0, :] = e_id
    p_ref[0, :] = p_top[:, 0]


def _route_sc_kernel(eid_hbm, p_hbm, se_hbm, st_hbm, sp_hbm,
                     eid_v, p_v, se_v, st_v, sp_v):
    wid = lax.axis_index("s") * 2 + lax.axis_index("c")

    @pl.when(wid == 0)
    def _():
        pltpu.sync_copy(eid_hbm, eid_v)
        pltpu.sync_copy(p_hbm, p_v)
        e0 = eid_v[pl.ds(0, 16)]
        e1 = eid_v[pl.ds(16, 16)]
        io = lax.iota(jnp.int32, 16)
        key0 = e0 * N + io
        key1 = e1 * N + io + 16
        r0 = jnp.zeros((16,), jnp.int32)
        r1 = jnp.zeros((16,), jnp.int32)
        for lane in range(16):
            kb0 = jnp.full((16,), key0[lane], jnp.int32)
            c0 = (plsc.all_reduce_population_count(key0 < kb0)
                  + plsc.all_reduce_population_count(key1 < kb0))
            r0 = jnp.where(io == lane, c0, r0)
            kb1 = jnp.full((16,), key1[lane], jnp.int32)
            c1 = (plsc.all_reduce_population_count(key0 < kb1)
                  + plsc.all_reduce_population_count(key1 < kb1))
            r1 = jnp.where(io == lane, c1, r1)
        p0 = p_v[pl.ds(0, 16)]
        p1 = p_v[pl.ds(16, 16)]
        plsc.store_scatter(se_v, [r0], e0)
        plsc.store_scatter(se_v, [r1], e1)
        plsc.store_scatter(st_v, [r0], io)
        plsc.store_scatter(st_v, [r1], io + 16)
        plsc.store_scatter(sp_v, [r0], p0)
        plsc.store_scatter(sp_v, [r1], p1)
        pltpu.sync_copy(se_v, se_hbm)
        pltpu.sync_copy(st_v, st_hbm)
        pltpu.sync_copy(sp_v, sp_hbm)


_route_sc = functools.partial(
    pl.kernel,
    out_type=(
        jax.ShapeDtypeStruct((N,), jnp.int32),
        jax.ShapeDtypeStruct((N,), jnp.int32),
        jax.ShapeDtypeStruct((N,), jnp.float32),
    ),
    mesh=plsc.VectorSubcoreMesh(core_axis_name="c", subcore_axis_name="s",
                                num_cores=1, num_subcores=1),
    compiler_params=pltpu.CompilerParams(needs_layout_passes=False),
    scratch_types=[
        pltpu.VMEM((N,), jnp.int32),
        pltpu.VMEM((N,), jnp.float32),
        pltpu.VMEM((N,), jnp.int32),
        pltpu.VMEM((N,), jnp.int32),
        pltpu.VMEM((N,), jnp.float32),
    ],
)(_route_sc_kernel)


def _ffn_kernel(se_ref, st_ref, sp_ref, x_ref, w1_ref, w3_ref, w2_ref, o_ref):
    del se_ref, st_ref
    i = pl.program_id(0)
    x = x_ref[0]
    w1 = w1_ref[0]
    w3 = w3_ref[0]
    w2 = w2_ref[0]
    h1 = jax.lax.dot_general(x, w1, (((1,), (1,)), ((), ())),
                             preferred_element_type=jnp.float32)
    h3 = jax.lax.dot_general(x, w3, (((1,), (1,)), ((), ())),
                             preferred_element_type=jnp.float32)
    h = jax.nn.silu(h1) * h3
    y = jax.lax.dot_general(h, w2, (((1,), (1,)), ((), ())),
                            preferred_element_type=jnp.float32)
    o_ref[0] = y * sp_ref[i]


def kernel(x, gate_w, w1, w2, w3):
    orig_shape = x.shape
    xf = x.reshape(-1, orig_shape[-1])

    eid, p = pl.pallas_call(
        _gate_kernel,
        out_shape=(
            jax.ShapeDtypeStruct((1, N), jnp.int32),
            jax.ShapeDtypeStruct((1, N), jnp.float32),
        ),
    )(xf, gate_w)

    se1, st1, sp = _route_sc(eid.reshape(N), p.reshape(N))

    x3 = xf.reshape(N, 1, D)

    grid_spec = pltpu.PrefetchScalarGridSpec(
        num_scalar_prefetch=3,
        grid=(N,),
        in_specs=[
            pl.BlockSpec((1, 1, D), lambda i, se_r, st_r, sp_r: (st_r[i], 0, 0)),
            pl.BlockSpec((1, H, D), lambda i, se_r, st_r, sp_r: (se_r[i], 0, 0)),
            pl.BlockSpec((1, H, D), lambda i, se_r, st_r, sp_r: (se_r[i], 0, 0)),
            pl.BlockSpec((1, D, H), lambda i, se_r, st_r, sp_r: (se_r[i], 0, 0)),
        ],
        out_specs=pl.BlockSpec((1, 1, D),
                               lambda i, se_r, st_r, sp_r: (st_r[i], 0, 0)),
    )

    y = pl.pallas_call(
        _ffn_kernel,
        grid_spec=grid_spec,
        out_shape=jax.ShapeDtypeStruct((N, 1, D), jnp.float32),
        compiler_params=pltpu.CompilerParams(
            vmem_limit_bytes=100 * 1024 * 1024,
        ),
    )(se1, st1, sp, x3, w1, w3, w2)

    return y.reshape(orig_shape)
